# baseline (speedup 1.0000x reference)
import functools

import jax
import jax.numpy as jnp
from jax import lax
from jax.experimental import pallas as pl
from jax.experimental.pallas import tpu as pltpu
from jax.experimental.pallas import tpu_sc as plsc

BATCH = 16384
EMBED_DIM = 32
HIDDEN_DIM = 64
FEAT_PAD = 8
SUPER = 128
PACK = SUPER // EMBED_DIM

_NC = 2
_NS = 16
_NW = _NC * _NS
_B_PER_W = BATCH // _NW


def _sc_gather(user_table_sr, item_table_sr, uidx4, iidx4):
    mesh = plsc.VectorSubcoreMesh(core_axis_name="c", subcore_axis_name="s")

    @functools.partial(
        pl.kernel,
        mesh=mesh,
        out_type=[
            jax.ShapeDtypeStruct((BATCH, SUPER), jnp.float32),
            jax.ShapeDtypeStruct((BATCH, SUPER), jnp.float32),
        ],
        scratch_types=[
            pltpu.VMEM((_B_PER_W,), jnp.int32),
            pltpu.VMEM((_B_PER_W, SUPER), jnp.float32),
            pltpu.SemaphoreType.DMA,
        ],
    )
    def gather_kernel(ut_hbm, it_hbm, uidx_hbm, iidx_hbm, uout_hbm, iout_hbm,
                      idx_v, rows_v, sem):
        wid = lax.axis_index("s") * _NC + lax.axis_index("c")
        base = wid * _B_PER_W
        pltpu.sync_copy(uidx_hbm.at[pl.ds(base, _B_PER_W)], idx_v)
        pltpu.async_copy(ut_hbm.at[idx_v], rows_v, sem).wait()
        pltpu.sync_copy(rows_v, uout_hbm.at[pl.ds(base, _B_PER_W)])
        pltpu.sync_copy(iidx_hbm.at[pl.ds(base, _B_PER_W)], idx_v)
        pltpu.async_copy(it_hbm.at[idx_v], rows_v, sem).wait()
        pltpu.sync_copy(rows_v, iout_hbm.at[pl.ds(base, _B_PER_W)])

    return gather_kernel(user_table_sr, item_table_sr, uidx4, iidx4)


def _mlp_body(gu_ref, gi_ref, f_ref, su_ref, si_ref, w1u_ref, w1i_ref,
              w1f_ref, b1_ref, w2_ref, b2_ref, out_ref):
    blk = gu_ref.shape[0]
    group = lax.broadcasted_iota(jnp.int32, (blk, SUPER), 1) // EMBED_DIM
    u = jnp.where(group == su_ref[...], gu_ref[...], 0.0)
    it = jnp.where(group == si_ref[...], gi_ref[...], 0.0)
    h = (
        jnp.dot(u, w1u_ref[...], preferred_element_type=jnp.float32)
        + jnp.dot(it, w1i_ref[...], preferred_element_type=jnp.float32)
        + jnp.dot(f_ref[...], w1f_ref[...], preferred_element_type=jnp.float32)
        + b1_ref[...]
    )
    h = jnp.maximum(h, 0.0)
    out_ref[...] = jnp.sum(h * w2_ref[...], axis=1, keepdims=True) + b2_ref[...]


def _tc_mlp(gu, gi, features_p, sel_u, sel_i, w1u4, w1i4, w1f, b1, w2r, b2):
    block = 2048
    grid = (BATCH // block,)
    const = lambda i: (0, 0)
    row = lambda i: (i, 0)
    return pl.pallas_call(
        _mlp_body,
        grid=grid,
        in_specs=[
            pl.BlockSpec((block, SUPER), row),
            pl.BlockSpec((block, SUPER), row),
            pl.BlockSpec((block, FEAT_PAD), row),
            pl.BlockSpec((block, 1), row),
            pl.BlockSpec((block, 1), row),
            pl.BlockSpec((SUPER, HIDDEN_DIM), const),
            pl.BlockSpec((SUPER, HIDDEN_DIM), const),
            pl.BlockSpec((FEAT_PAD, HIDDEN_DIM), const),
            pl.BlockSpec((1, HIDDEN_DIM), const),
            pl.BlockSpec((1, HIDDEN_DIM), const),
            pl.BlockSpec((1, 1), const),
        ],
        out_specs=pl.BlockSpec((block, 1), row),
        out_shape=jax.ShapeDtypeStruct((BATCH, 1), jnp.float32),
    )(gu, gi, features_p, sel_u, sel_i, w1u4, w1i4, w1f, b1, w2r, b2)


def kernel(user, item, features, user_table, item_table, W1, b1, W2, b2):
    user = user.astype(jnp.int32)
    item = item.astype(jnp.int32)

    ut_sr = user_table.reshape(user_table.shape[0] // PACK, SUPER)
    it_sr = item_table.reshape(item_table.shape[0] // PACK, SUPER)
    gu, gi = _sc_gather(ut_sr, it_sr, user // PACK, item // PACK)

    sel_u = (user % PACK).reshape(BATCH, 1)
    sel_i = (item % PACK).reshape(BATCH, 1)
    features_p = jnp.pad(features, ((0, 0), (0, FEAT_PAD - features.shape[1])))
    w1u = W1[:EMBED_DIM]
    w1i = W1[EMBED_DIM:2 * EMBED_DIM]
    w1u4 = jnp.concatenate([w1u] * PACK, axis=0)
    w1i4 = jnp.concatenate([w1i] * PACK, axis=0)
    w1f = jnp.pad(W1[2 * EMBED_DIM:], ((0, FEAT_PAD - 3), (0, 0)))
    b1r = b1.reshape(1, HIDDEN_DIM)
    w2r = W2.reshape(1, HIDDEN_DIM)
    b2r = b2.reshape(1, 1)

    out = _tc_mlp(gu, gi, features_p, sel_u, sel_i, w1u4, w1i4, w1f,
                  b1r, w2r, b2r)
    return out.reshape(BATCH)


# speedup vs baseline: 1.0004x; 1.0004x over previous
import functools

import jax
import jax.numpy as jnp
from jax import lax
from jax.experimental import pallas as pl
from jax.experimental.pallas import tpu as pltpu
from jax.experimental.pallas import tpu_sc as plsc

BATCH = 16384
EMBED_DIM = 32
HIDDEN_DIM = 64
FEAT_PAD = 8
SUPER = 128
PACK = SUPER // EMBED_DIM

_NC = 2
_NS = 16
_NW = _NC * _NS
_B_PER_W = BATCH // _NW


def _sc_gather(user_table_sr, item_table_sr, uidx4, iidx4):
    mesh = plsc.VectorSubcoreMesh(core_axis_name="c", subcore_axis_name="s")

    @functools.partial(
        pl.kernel,
        mesh=mesh,
        compiler_params=pltpu.CompilerParams(use_tc_tiling_on_sc=True),
        out_type=[
            jax.ShapeDtypeStruct((BATCH, SUPER), jnp.float32),
            jax.ShapeDtypeStruct((BATCH, SUPER), jnp.float32),
        ],
        scratch_types=[
            pltpu.VMEM((_B_PER_W,), jnp.int32),
            pltpu.VMEM((_B_PER_W, SUPER), jnp.float32),
            pltpu.SemaphoreType.DMA,
        ],
    )
    def gather_kernel(ut_hbm, it_hbm, uidx_hbm, iidx_hbm, uout_hbm, iout_hbm,
                      idx_v, rows_v, sem):
        wid = lax.axis_index("s") * _NC + lax.axis_index("c")
        base = wid * _B_PER_W
        pltpu.sync_copy(uidx_hbm.at[pl.ds(base, _B_PER_W)], idx_v)
        pltpu.async_copy(ut_hbm.at[idx_v], rows_v, sem).wait()
        pltpu.sync_copy(rows_v, uout_hbm.at[pl.ds(base, _B_PER_W)])
        pltpu.sync_copy(iidx_hbm.at[pl.ds(base, _B_PER_W)], idx_v)
        pltpu.async_copy(it_hbm.at[idx_v], rows_v, sem).wait()
        pltpu.sync_copy(rows_v, iout_hbm.at[pl.ds(base, _B_PER_W)])

    return gather_kernel(user_table_sr, item_table_sr, uidx4, iidx4)


def _mlp_body(gu_ref, gi_ref, f_ref, su_ref, si_ref, w1u_ref, w1i_ref,
              w1f_ref, b1_ref, w2_ref, b2_ref, out_ref):
    blk = gu_ref.shape[0]
    group = lax.broadcasted_iota(jnp.int32, (blk, SUPER), 1) // EMBED_DIM
    u = jnp.where(group == su_ref[...], gu_ref[...], 0.0)
    it = jnp.where(group == si_ref[...], gi_ref[...], 0.0)
    h = (
        jnp.dot(u, w1u_ref[...], preferred_element_type=jnp.float32)
        + jnp.dot(it, w1i_ref[...], preferred_element_type=jnp.float32)
        + jnp.dot(f_ref[...], w1f_ref[...], preferred_element_type=jnp.float32)
        + b1_ref[...]
    )
    h = jnp.maximum(h, 0.0)
    out_ref[...] = jnp.sum(h * w2_ref[...], axis=1, keepdims=True) + b2_ref[...]


def _tc_mlp(gu, gi, features_p, sel_u, sel_i, w1u4, w1i4, w1f, b1, w2r, b2):
    block = 2048
    grid = (BATCH // block,)
    const = lambda i: (0, 0)
    row = lambda i: (i, 0)
    return pl.pallas_call(
        _mlp_body,
        grid=grid,
        in_specs=[
            pl.BlockSpec((block, SUPER), row),
            pl.BlockSpec((block, SUPER), row),
            pl.BlockSpec((block, FEAT_PAD), row),
            pl.BlockSpec((block, 1), row),
            pl.BlockSpec((block, 1), row),
            pl.BlockSpec((SUPER, HIDDEN_DIM), const),
            pl.BlockSpec((SUPER, HIDDEN_DIM), const),
            pl.BlockSpec((FEAT_PAD, HIDDEN_DIM), const),
            pl.BlockSpec((1, HIDDEN_DIM), const),
            pl.BlockSpec((1, HIDDEN_DIM), const),
            pl.BlockSpec((1, 1), const),
        ],
        out_specs=pl.BlockSpec((block, 1), row),
        out_shape=jax.ShapeDtypeStruct((BATCH, 1), jnp.float32),
    )(gu, gi, features_p, sel_u, sel_i, w1u4, w1i4, w1f, b1, w2r, b2)


def kernel(user, item, features, user_table, item_table, W1, b1, W2, b2):
    user = user.astype(jnp.int32)
    item = item.astype(jnp.int32)

    ut_sr = user_table.reshape(user_table.shape[0] // PACK, SUPER)
    it_sr = item_table.reshape(item_table.shape[0] // PACK, SUPER)
    gu, gi = _sc_gather(ut_sr, it_sr, user // PACK, item // PACK)

    sel_u = (user % PACK).reshape(BATCH, 1)
    sel_i = (item % PACK).reshape(BATCH, 1)
    features_p = jnp.pad(features, ((0, 0), (0, FEAT_PAD - features.shape[1])))
    w1u = W1[:EMBED_DIM]
    w1i = W1[EMBED_DIM:2 * EMBED_DIM]
    w1u4 = jnp.concatenate([w1u] * PACK, axis=0)
    w1i4 = jnp.concatenate([w1i] * PACK, axis=0)
    w1f = jnp.pad(W1[2 * EMBED_DIM:], ((0, FEAT_PAD - 3), (0, 0)))
    b1r = b1.reshape(1, HIDDEN_DIM)
    w2r = W2.reshape(1, HIDDEN_DIM)
    b2r = b2.reshape(1, 1)

    out = _tc_mlp(gu, gi, features_p, sel_u, sel_i, w1u4, w1i4, w1f,
                  b1r, w2r, b2r)
    return out.reshape(BATCH)


# speedup vs baseline: 1.6716x; 1.6710x over previous
import functools

import jax
import jax.numpy as jnp
from jax import lax
from jax.experimental import pallas as pl
from jax.experimental.pallas import tpu as pltpu
from jax.experimental.pallas import tpu_sc as plsc

BATCH = 16384
EMBED_DIM = 32
HIDDEN_DIM = 64
FEAT_PAD = 8
SUPER = 128
PACK = SUPER // EMBED_DIM

S_U, C_U = 251904, 2048
S_I, C_I = 25088, 3584

_NC = 2
_NS = 16
_NW = _NC * _NS
_B_PER_W = BATCH // _NW


def _relayout_body(a0_ref, a1_ref, a2_ref, a3_ref, out_ref):
    out_ref[...] = jnp.concatenate(
        [jnp.transpose(r[...]) for r in (a0_ref, a1_ref, a2_ref, a3_ref)],
        axis=1)


def _relayout(tt, slab, chunk):
    steps = slab // chunk
    last_block = (tt.shape[1] + chunk - 1) // chunk - 1

    def in_spec(a):
        return pl.BlockSpec(
            (EMBED_DIM, chunk),
            lambda k, a=a: (0, jnp.minimum(a * steps + k, last_block)))

    return pl.pallas_call(
        _relayout_body,
        grid=(steps,),
        in_specs=[in_spec(0), in_spec(1), in_spec(2), in_spec(3)],
        out_specs=pl.BlockSpec((chunk, SUPER), lambda k: (k, 0)),
        out_shape=jax.ShapeDtypeStruct((slab, SUPER), jnp.float32),
    )(tt, tt, tt, tt)


def _sc_gather(table_sr, idx):
    mesh = plsc.VectorSubcoreMesh(core_axis_name="c", subcore_axis_name="s")

    @functools.partial(
        pl.kernel,
        mesh=mesh,
        compiler_params=pltpu.CompilerParams(use_tc_tiling_on_sc=True),
        out_type=jax.ShapeDtypeStruct((BATCH, SUPER), jnp.float32),
        scratch_types=[
            pltpu.VMEM((_B_PER_W,), jnp.int32),
            pltpu.VMEM((_B_PER_W, SUPER), jnp.float32),
            pltpu.SemaphoreType.DMA,
        ],
    )
    def gather_kernel(tab_hbm, idx_hbm, out_hbm, idx_v, rows_v, sem):
        wid = lax.axis_index("s") * _NC + lax.axis_index("c")
        base = wid * _B_PER_W
        pltpu.sync_copy(idx_hbm.at[pl.ds(base, _B_PER_W)], idx_v)
        pltpu.async_copy(tab_hbm.at[idx_v], rows_v, sem).wait()
        pltpu.sync_copy(rows_v, out_hbm.at[pl.ds(base, _B_PER_W)])

    return gather_kernel(table_sr, idx)


def _mlp_body(gu_ref, gi_ref, f_ref, su_ref, si_ref, w1u_ref, w1i_ref,
              w1f_ref, b1_ref, w2_ref, b2_ref, out_ref):
    blk = gu_ref.shape[0]
    group = lax.broadcasted_iota(jnp.int32, (blk, SUPER), 1) // EMBED_DIM
    u = jnp.where(group == su_ref[...], gu_ref[...], 0.0)
    it = jnp.where(group == si_ref[...], gi_ref[...], 0.0)
    h = (
        jnp.dot(u, w1u_ref[...], preferred_element_type=jnp.float32)
        + jnp.dot(it, w1i_ref[...], preferred_element_type=jnp.float32)
        + jnp.dot(f_ref[...], w1f_ref[...], preferred_element_type=jnp.float32)
        + b1_ref[...]
    )
    h = jnp.maximum(h, 0.0)
    out_ref[...] = jnp.sum(h * w2_ref[...], axis=1, keepdims=True) + b2_ref[...]


def _tc_mlp(gu, gi, features_p, sel_u, sel_i, w1u4, w1i4, w1f, b1, w2r, b2):
    block = 2048
    grid = (BATCH // block,)
    const = lambda i: (0, 0)
    row = lambda i: (i, 0)
    return pl.pallas_call(
        _mlp_body,
        grid=grid,
        in_specs=[
            pl.BlockSpec((block, SUPER), row),
            pl.BlockSpec((block, SUPER), row),
            pl.BlockSpec((block, FEAT_PAD), row),
            pl.BlockSpec((block, 1), row),
            pl.BlockSpec((block, 1), row),
            pl.BlockSpec((SUPER, HIDDEN_DIM), const),
            pl.BlockSpec((SUPER, HIDDEN_DIM), const),
            pl.BlockSpec((FEAT_PAD, HIDDEN_DIM), const),
            pl.BlockSpec((1, HIDDEN_DIM), const),
            pl.BlockSpec((1, HIDDEN_DIM), const),
            pl.BlockSpec((1, 1), const),
        ],
        out_specs=pl.BlockSpec((block, 1), row),
        out_shape=jax.ShapeDtypeStruct((BATCH, 1), jnp.float32),
    )(gu, gi, features_p, sel_u, sel_i, w1u4, w1i4, w1f, b1, w2r, b2)


def kernel(user, item, features, user_table, item_table, W1, b1, W2, b2):
    user = user.astype(jnp.int32)
    item = item.astype(jnp.int32)

    it_sr = _relayout(item_table.T, S_I, C_I)
    gi = _sc_gather(it_sr, item % S_I)
    ut_sr = _relayout(user_table.T, S_U, C_U)
    gu = _sc_gather(ut_sr, user % S_U)

    sel_u = (user // S_U).reshape(BATCH, 1)
    sel_i = (item // S_I).reshape(BATCH, 1)
    features_p = jnp.pad(features, ((0, 0), (0, FEAT_PAD - features.shape[1])))
    w1u = W1[:EMBED_DIM]
    w1i = W1[EMBED_DIM:2 * EMBED_DIM]
    w1u4 = jnp.concatenate([w1u] * PACK, axis=0)
    w1i4 = jnp.concatenate([w1i] * PACK, axis=0)
    w1f = jnp.pad(W1[2 * EMBED_DIM:], ((0, FEAT_PAD - 3), (0, 0)))
    b1r = b1.reshape(1, HIDDEN_DIM)
    w2r = W2.reshape(1, HIDDEN_DIM)
    b2r = b2.reshape(1, 1)

    out = _tc_mlp(gu, gi, features_p, sel_u, sel_i, w1u4, w1i4, w1f,
                  b1r, w2r, b2r)
    return out.reshape(BATCH)


# speedup vs baseline: 2.6661x; 1.5950x over previous
import functools

import jax
import jax.numpy as jnp
from jax import lax
from jax.experimental import pallas as pl
from jax.experimental.pallas import tpu as pltpu
from jax.experimental.pallas import tpu_sc as plsc

BATCH = 16384
EMBED_DIM = 32
HIDDEN_DIM = 64
FEAT_PAD = 8
SUPER = 128
PACK = SUPER // EMBED_DIM

S_U, C_U = 251904, 2048
S_I, C_I = 25088, 3584

_NC = 2
_NS = 16
_NW = _NC * _NS
_B_PER_W = BATCH // _NW


def _relayout_body(a0_ref, a1_ref, a2_ref, a3_ref, out_ref):
    stacked = jnp.concatenate(
        [r[...] for r in (a0_ref, a1_ref, a2_ref, a3_ref)], axis=0)
    out_ref[...] = jnp.transpose(stacked)


def _relayout(tt, slab, chunk):
    steps = slab // chunk
    last_block = (tt.shape[1] + chunk - 1) // chunk - 1

    def in_spec(a):
        return pl.BlockSpec(
            (EMBED_DIM, chunk),
            lambda k, a=a: (0, jnp.minimum(a * steps + k, last_block)))

    return pl.pallas_call(
        _relayout_body,
        grid=(steps,),
        in_specs=[in_spec(0), in_spec(1), in_spec(2), in_spec(3)],
        out_specs=pl.BlockSpec((chunk, SUPER), lambda k: (k, 0)),
        out_shape=jax.ShapeDtypeStruct((slab, SUPER), jnp.float32),
    )(tt, tt, tt, tt)


def _sc_gather(table_sr, idx):
    mesh = plsc.VectorSubcoreMesh(core_axis_name="c", subcore_axis_name="s")

    @functools.partial(
        pl.kernel,
        mesh=mesh,
        compiler_params=pltpu.CompilerParams(use_tc_tiling_on_sc=True),
        out_type=jax.ShapeDtypeStruct((BATCH, SUPER), jnp.float32),
        scratch_types=[
            pltpu.VMEM((_B_PER_W,), jnp.int32),
            pltpu.VMEM((_B_PER_W, SUPER), jnp.float32),
            pltpu.SemaphoreType.DMA,
        ],
    )
    def gather_kernel(tab_hbm, idx_hbm, out_hbm, idx_v, rows_v, sem):
        wid = lax.axis_index("s") * _NC + lax.axis_index("c")
        base = wid * _B_PER_W
        pltpu.sync_copy(idx_hbm.at[pl.ds(base, _B_PER_W)], idx_v)
        pltpu.async_copy(tab_hbm.at[idx_v], rows_v, sem).wait()
        pltpu.sync_copy(rows_v, out_hbm.at[pl.ds(base, _B_PER_W)])

    return gather_kernel(table_sr, idx)


def _mlp_body(gu_ref, gi_ref, f_ref, su_ref, si_ref, w1u_ref, w1i_ref,
              w1f_ref, b1_ref, w2_ref, b2_ref, out_ref):
    blk = gu_ref.shape[0]
    group = lax.broadcasted_iota(jnp.int32, (blk, SUPER), 1) // EMBED_DIM
    u = jnp.where(group == su_ref[...], gu_ref[...], 0.0)
    it = jnp.where(group == si_ref[...], gi_ref[...], 0.0)
    h = (
        jnp.dot(u, w1u_ref[...], preferred_element_type=jnp.float32)
        + jnp.dot(it, w1i_ref[...], preferred_element_type=jnp.float32)
        + jnp.dot(f_ref[...], w1f_ref[...], preferred_element_type=jnp.float32)
        + b1_ref[...]
    )
    h = jnp.maximum(h, 0.0)
    out_ref[...] = jnp.sum(h * w2_ref[...], axis=1, keepdims=True) + b2_ref[...]


def _tc_mlp(gu, gi, features_p, sel_u, sel_i, w1u4, w1i4, w1f, b1, w2r, b2):
    block = 2048
    grid = (BATCH // block,)
    const = lambda i: (0, 0)
    row = lambda i: (i, 0)
    return pl.pallas_call(
        _mlp_body,
        grid=grid,
        in_specs=[
            pl.BlockSpec((block, SUPER), row),
            pl.BlockSpec((block, SUPER), row),
            pl.BlockSpec((block, FEAT_PAD), row),
            pl.BlockSpec((block, 1), row),
            pl.BlockSpec((block, 1), row),
            pl.BlockSpec((SUPER, HIDDEN_DIM), const),
            pl.BlockSpec((SUPER, HIDDEN_DIM), const),
            pl.BlockSpec((FEAT_PAD, HIDDEN_DIM), const),
            pl.BlockSpec((1, HIDDEN_DIM), const),
            pl.BlockSpec((1, HIDDEN_DIM), const),
            pl.BlockSpec((1, 1), const),
        ],
        out_specs=pl.BlockSpec((block, 1), row),
        out_shape=jax.ShapeDtypeStruct((BATCH, 1), jnp.float32),
    )(gu, gi, features_p, sel_u, sel_i, w1u4, w1i4, w1f, b1, w2r, b2)


def kernel(user, item, features, user_table, item_table, W1, b1, W2, b2):
    user = user.astype(jnp.int32)
    item = item.astype(jnp.int32)

    it_sr = _relayout(item_table.T, S_I, C_I)
    gi = _sc_gather(it_sr, item % S_I)
    ut_sr = _relayout(user_table.T, S_U, C_U)
    gu = _sc_gather(ut_sr, user % S_U)

    sel_u = (user // S_U).reshape(BATCH, 1)
    sel_i = (item // S_I).reshape(BATCH, 1)
    features_p = jnp.pad(features, ((0, 0), (0, FEAT_PAD - features.shape[1])))
    w1u = W1[:EMBED_DIM]
    w1i = W1[EMBED_DIM:2 * EMBED_DIM]
    w1u4 = jnp.concatenate([w1u] * PACK, axis=0)
    w1i4 = jnp.concatenate([w1i] * PACK, axis=0)
    w1f = jnp.pad(W1[2 * EMBED_DIM:], ((0, FEAT_PAD - 3), (0, 0)))
    b1r = b1.reshape(1, HIDDEN_DIM)
    w2r = W2.reshape(1, HIDDEN_DIM)
    b2r = b2.reshape(1, 1)

    out = _tc_mlp(gu, gi, features_p, sel_u, sel_i, w1u4, w1i4, w1f,
                  b1r, w2r, b2r)
    return out.reshape(BATCH)


# speedup vs baseline: 3.7150x; 1.3934x over previous
import functools

import jax
import jax.numpy as jnp
from jax import lax
from jax.experimental import pallas as pl
from jax.experimental.pallas import tpu as pltpu
from jax.experimental.pallas import tpu_sc as plsc

BATCH = 16384
EMBED_DIM = 32
HIDDEN_DIM = 64
FEAT_PAD = 8
SUPER = 128
PACK = 8
GROUPS = 4

CHUNK = 4096
S_U = 126976
S_I = 16384
STEPS_U = S_U // CHUNK
STEPS_I = S_I // CHUNK
ROWS_OUT = S_U + S_I

_NC = 2
_NS = 16
_NW = _NC * _NS
_IDX_N = 2 * BATCH
_B_PER_W = _IDX_N // _NW


def _pack_bf16_pair(ta, tb):
    ua = lax.bitcast_convert_type(ta, jnp.uint32)
    ub = lax.bitcast_convert_type(tb, jnp.uint32)
    ra = (ua + jnp.uint32(0x7FFF) + ((ua >> 16) & jnp.uint32(1))) >> 16
    rb = (ub + jnp.uint32(0x7FFF) + ((ub >> 16) & jnp.uint32(1))) >> 16
    return lax.bitcast_convert_type((rb << 16) | ra, jnp.int32)


def _relayout_body(*refs):
    u_slabs, i_slabs, out_ref = refs[:PACK], refs[PACK:2 * PACK], refs[-1]
    k = pl.program_id(0)

    def emit(slabs):
        a = jnp.concatenate([slabs[g][...] for g in (0, 2, 4, 6)], axis=0)
        b = jnp.concatenate([slabs[g][...] for g in (1, 3, 5, 7)], axis=0)
        out_ref[...] = _pack_bf16_pair(jnp.transpose(a), jnp.transpose(b))

    @pl.when(k < STEPS_U)
    def _():
        emit(u_slabs)

    @pl.when(k >= STEPS_U)
    def _():
        emit(i_slabs)


def _relayout(tt_u, tt_i):
    last_u = (tt_u.shape[1] + CHUNK - 1) // CHUNK - 1
    last_i = (tt_i.shape[1] + CHUNK - 1) // CHUNK - 1

    def u_spec(a):
        return pl.BlockSpec(
            (EMBED_DIM, CHUNK),
            lambda k, a=a: (0, jnp.minimum(
                a * STEPS_U + jnp.minimum(k, STEPS_U - 1), last_u)))

    def i_spec(a):
        return pl.BlockSpec(
            (EMBED_DIM, CHUNK),
            lambda k, a=a: (0, jnp.minimum(
                a * STEPS_I + jnp.clip(k - STEPS_U, 0, STEPS_I - 1), last_i)))

    return pl.pallas_call(
        _relayout_body,
        grid=(STEPS_U + STEPS_I,),
        in_specs=[u_spec(a) for a in range(PACK)]
        + [i_spec(a) for a in range(PACK)],
        out_specs=pl.BlockSpec((CHUNK, SUPER), lambda k: (k, 0)),
        out_shape=jax.ShapeDtypeStruct((ROWS_OUT, SUPER), jnp.int32),
    )(*([tt_u] * PACK + [tt_i] * PACK))


def _sc_gather(table_sr, idx):
    mesh = plsc.VectorSubcoreMesh(core_axis_name="c", subcore_axis_name="s")
    half = BATCH // _NW

    @functools.partial(
        pl.kernel,
        mesh=mesh,
        compiler_params=pltpu.CompilerParams(use_tc_tiling_on_sc=True),
        out_type=jax.ShapeDtypeStruct((BATCH, SUPER), jnp.int32),
        scratch_types=[
            pltpu.VMEM((half,), jnp.int32),
            pltpu.VMEM((half, SUPER), jnp.int32),
            pltpu.SemaphoreType.DMA,
        ],
    )
    def gather_kernel(tab_hbm, idx_hbm, out_hbm, idx_v, rows_v, sem):
        wid = lax.axis_index("s") * _NC + lax.axis_index("c")
        base = wid * half
        pltpu.sync_copy(idx_hbm.at[pl.ds(base, half)], idx_v)
        pltpu.async_copy(tab_hbm.at[idx_v], rows_v, sem).wait()
        pltpu.sync_copy(rows_v, out_hbm.at[pl.ds(base, half)])

    return gather_kernel(table_sr, idx)


def _mlp_body(gu_ref, gi_ref, f_ref, su_ref, si_ref, w1u_ref, w1i_ref,
              w1f_ref, b1_ref, w2_ref, b2_ref, out_ref):
    blk = gu_ref.shape[0]
    group = lax.broadcasted_iota(jnp.int32, (blk, SUPER), 1) // EMBED_DIM

    def unpack(g_ref, sel_ref):
        g = g_ref[...]
        sel = sel_ref[...]
        grp = sel >> 1
        half = sel & 1
        bits = jnp.where(half == 0, g << 16, g & jnp.int32(-65536))
        f = lax.bitcast_convert_type(bits, jnp.float32)
        return jnp.where(group == grp, f, 0.0).astype(jnp.bfloat16)

    u = unpack(gu_ref, su_ref)
    it = unpack(gi_ref, si_ref)
    h = (
        jnp.dot(u, w1u_ref[...], preferred_element_type=jnp.float32)
        + jnp.dot(it, w1i_ref[...], preferred_element_type=jnp.float32)
        + jnp.dot(f_ref[...], w1f_ref[...], preferred_element_type=jnp.float32)
        + b1_ref[...]
    )
    h = jnp.maximum(h, 0.0)
    out_ref[...] = jnp.sum(h * w2_ref[...], axis=1, keepdims=True) + b2_ref[...]


def _tc_mlp(gu, gi, features_p, sel_u, sel_i, w1u4, w1i4, w1f, b1, w2r, b2):
    block = 2048
    grid = (BATCH // block,)
    const = lambda i: (0, 0)
    row = lambda i: (i, 0)
    return pl.pallas_call(
        _mlp_body,
        grid=grid,
        in_specs=[
            pl.BlockSpec((block, SUPER), row),
            pl.BlockSpec((block, SUPER), row),
            pl.BlockSpec((block, FEAT_PAD), row),
            pl.BlockSpec((block, 1), row),
            pl.BlockSpec((block, 1), row),
            pl.BlockSpec((SUPER, HIDDEN_DIM), const),
            pl.BlockSpec((SUPER, HIDDEN_DIM), const),
            pl.BlockSpec((FEAT_PAD, HIDDEN_DIM), const),
            pl.BlockSpec((1, HIDDEN_DIM), const),
            pl.BlockSpec((1, HIDDEN_DIM), const),
            pl.BlockSpec((1, 1), const),
        ],
        out_specs=pl.BlockSpec((block, 1), row),
        out_shape=jax.ShapeDtypeStruct((BATCH, 1), jnp.float32),
    )(gu, gi, features_p, sel_u, sel_i, w1u4, w1i4, w1f, b1, w2r, b2)


def kernel(user, item, features, user_table, item_table, W1, b1, W2, b2):
    user = user.astype(jnp.int32)
    item = item.astype(jnp.int32)

    table_sr = _relayout(user_table.T, item_table.T)
    gu = _sc_gather(table_sr, user % S_U)
    gi = _sc_gather(table_sr, S_U + item % S_I)

    sel_u = (user // S_U).reshape(BATCH, 1)
    sel_i = (item // S_I).reshape(BATCH, 1)
    features_p = jnp.pad(features, ((0, 0), (0, FEAT_PAD - features.shape[1])))
    w1u = W1[:EMBED_DIM].astype(jnp.bfloat16)
    w1i = W1[EMBED_DIM:2 * EMBED_DIM].astype(jnp.bfloat16)
    w1u4 = jnp.concatenate([w1u] * GROUPS, axis=0)
    w1i4 = jnp.concatenate([w1i] * GROUPS, axis=0)
    w1f = jnp.pad(W1[2 * EMBED_DIM:], ((0, FEAT_PAD - 3), (0, 0)))
    b1r = b1.reshape(1, HIDDEN_DIM)
    w2r = W2.reshape(1, HIDDEN_DIM)
    b2r = b2.reshape(1, 1)

    out = _tc_mlp(gu, gi, features_p, sel_u, sel_i, w1u4, w1i4, w1f,
                  b1r, w2r, b2r)
    return out.reshape(BATCH)


# speedup vs baseline: 3.9203x; 1.0553x over previous
import functools

import jax
import jax.numpy as jnp
from jax import lax
from jax.experimental import pallas as pl
from jax.experimental.pallas import tpu as pltpu
from jax.experimental.pallas import tpu_sc as plsc

BATCH = 16384
EMBED_DIM = 32
HIDDEN_DIM = 64
FEAT_PAD = 8
SUPER = 128
PACK = 8
GROUPS = 4

CHUNK = 4096
S_U = 126976
S_I = 16384
STEPS_U = S_U // CHUNK
STEPS_I = S_I // CHUNK
ROWS_OUT = S_U + S_I

_NC = 2
_NS = 16
_NW = _NC * _NS
_IDX_N = 2 * BATCH
_B_PER_W = _IDX_N // _NW


def _pack_bf16_pair(ta, tb):
    ua = lax.bitcast_convert_type(ta, jnp.uint32)
    ub = lax.bitcast_convert_type(tb, jnp.uint32)
    ra = (ua + jnp.uint32(0x8000)) >> 16
    rb = (ub + jnp.uint32(0x8000)) & jnp.uint32(0xFFFF0000)
    return lax.bitcast_convert_type(rb | ra, jnp.int32)


def _relayout_body(*refs):
    u_slabs, i_slabs, out_ref = refs[:PACK], refs[PACK:2 * PACK], refs[-1]
    k = pl.program_id(0)

    def emit(slabs):
        a = jnp.concatenate([slabs[g][...] for g in (0, 2, 4, 6)], axis=0)
        b = jnp.concatenate([slabs[g][...] for g in (1, 3, 5, 7)], axis=0)
        out_ref[...] = jnp.transpose(_pack_bf16_pair(a, b))

    @pl.when(k < STEPS_U)
    def _():
        emit(u_slabs)

    @pl.when(k >= STEPS_U)
    def _():
        emit(i_slabs)


def _relayout(tt_u, tt_i):
    last_u = (tt_u.shape[1] + CHUNK - 1) // CHUNK - 1
    last_i = (tt_i.shape[1] + CHUNK - 1) // CHUNK - 1

    def u_spec(a):
        return pl.BlockSpec(
            (EMBED_DIM, CHUNK),
            lambda k, a=a: (0, jnp.minimum(
                a * STEPS_U + jnp.minimum(k, STEPS_U - 1), last_u)))

    def i_spec(a):
        return pl.BlockSpec(
            (EMBED_DIM, CHUNK),
            lambda k, a=a: (0, jnp.minimum(
                a * STEPS_I + jnp.clip(k - STEPS_U, 0, STEPS_I - 1), last_i)))

    return pl.pallas_call(
        _relayout_body,
        grid=(STEPS_U + STEPS_I,),
        in_specs=[u_spec(a) for a in range(PACK)]
        + [i_spec(a) for a in range(PACK)],
        out_specs=pl.BlockSpec((CHUNK, SUPER), lambda k: (k, 0)),
        out_shape=jax.ShapeDtypeStruct((ROWS_OUT, SUPER), jnp.int32),
    )(*([tt_u] * PACK + [tt_i] * PACK))


def _sc_gather(table_sr, idx):
    mesh = plsc.VectorSubcoreMesh(core_axis_name="c", subcore_axis_name="s")
    half = BATCH // _NW

    @functools.partial(
        pl.kernel,
        mesh=mesh,
        compiler_params=pltpu.CompilerParams(use_tc_tiling_on_sc=True),
        out_type=jax.ShapeDtypeStruct((BATCH, SUPER), jnp.int32),
        scratch_types=[
            pltpu.VMEM((half,), jnp.int32),
            pltpu.VMEM((half, SUPER), jnp.int32),
            pltpu.SemaphoreType.DMA,
        ],
    )
    def gather_kernel(tab_hbm, idx_hbm, out_hbm, idx_v, rows_v, sem):
        wid = lax.axis_index("s") * _NC + lax.axis_index("c")
        base = wid * half
        pltpu.sync_copy(idx_hbm.at[pl.ds(base, half)], idx_v)
        pltpu.async_copy(tab_hbm.at[idx_v], rows_v, sem).wait()
        pltpu.sync_copy(rows_v, out_hbm.at[pl.ds(base, half)])

    return gather_kernel(table_sr, idx)


def _mlp_body(gu_ref, gi_ref, f_ref, su_ref, si_ref, w1u_ref, w1i_ref,
              w1f_ref, b1_ref, w2_ref, b2_ref, out_ref):
    blk = gu_ref.shape[0]
    group = lax.broadcasted_iota(jnp.int32, (blk, SUPER), 1) // EMBED_DIM

    def unpack(g_ref, sel_ref):
        g = g_ref[...]
        sel = sel_ref[...]
        grp = sel >> 1
        half = sel & 1
        bits = jnp.where(half == 0, g << 16, g & jnp.int32(-65536))
        f = lax.bitcast_convert_type(bits, jnp.float32)
        return jnp.where(group == grp, f, 0.0).astype(jnp.bfloat16)

    u = unpack(gu_ref, su_ref)
    it = unpack(gi_ref, si_ref)
    h = (
        jnp.dot(u, w1u_ref[...], preferred_element_type=jnp.float32)
        + jnp.dot(it, w1i_ref[...], preferred_element_type=jnp.float32)
        + jnp.dot(f_ref[...], w1f_ref[...], preferred_element_type=jnp.float32)
        + b1_ref[...]
    )
    h = jnp.maximum(h, 0.0)
    out_ref[...] = jnp.sum(h * w2_ref[...], axis=1, keepdims=True) + b2_ref[...]


def _tc_mlp(gu, gi, features_p, sel_u, sel_i, w1u4, w1i4, w1f, b1, w2r, b2):
    block = 2048
    grid = (BATCH // block,)
    const = lambda i: (0, 0)
    row = lambda i: (i, 0)
    return pl.pallas_call(
        _mlp_body,
        grid=grid,
        in_specs=[
            pl.BlockSpec((block, SUPER), row),
            pl.BlockSpec((block, SUPER), row),
            pl.BlockSpec((block, FEAT_PAD), row),
            pl.BlockSpec((block, 1), row),
            pl.BlockSpec((block, 1), row),
            pl.BlockSpec((SUPER, HIDDEN_DIM), const),
            pl.BlockSpec((SUPER, HIDDEN_DIM), const),
            pl.BlockSpec((FEAT_PAD, HIDDEN_DIM), const),
            pl.BlockSpec((1, HIDDEN_DIM), const),
            pl.BlockSpec((1, HIDDEN_DIM), const),
            pl.BlockSpec((1, 1), const),
        ],
        out_specs=pl.BlockSpec((block, 1), row),
        out_shape=jax.ShapeDtypeStruct((BATCH, 1), jnp.float32),
    )(gu, gi, features_p, sel_u, sel_i, w1u4, w1i4, w1f, b1, w2r, b2)


def kernel(user, item, features, user_table, item_table, W1, b1, W2, b2):
    user = user.astype(jnp.int32)
    item = item.astype(jnp.int32)

    table_sr = _relayout(user_table.T, item_table.T)
    gu = _sc_gather(table_sr, user % S_U)
    gi = _sc_gather(table_sr, S_U + item % S_I)

    sel_u = (user // S_U).reshape(BATCH, 1)
    sel_i = (item // S_I).reshape(BATCH, 1)
    features_p = jnp.pad(features, ((0, 0), (0, FEAT_PAD - features.shape[1])))
    w1u = W1[:EMBED_DIM].astype(jnp.bfloat16)
    w1i = W1[EMBED_DIM:2 * EMBED_DIM].astype(jnp.bfloat16)
    w1u4 = jnp.concatenate([w1u] * GROUPS, axis=0)
    w1i4 = jnp.concatenate([w1i] * GROUPS, axis=0)
    w1f = jnp.pad(W1[2 * EMBED_DIM:], ((0, FEAT_PAD - 3), (0, 0)))
    b1r = b1.reshape(1, HIDDEN_DIM)
    w2r = W2.reshape(1, HIDDEN_DIM)
    b2r = b2.reshape(1, 1)

    out = _tc_mlp(gu, gi, features_p, sel_u, sel_i, w1u4, w1i4, w1f,
                  b1r, w2r, b2r)
    return out.reshape(BATCH)
